# Initial kernel scaffold; baseline (speedup 1.0000x reference)
import jax
import jax.numpy as jnp
from jax.experimental import pallas as pl


def kernel(features, Wv, bv, Wu, bu, w_att, b_att):
    raise NotImplementedError("write your pallas kernel here")


# speedup vs baseline: 2.7095x; 2.7095x over previous
import functools

import jax
import jax.numpy as jnp
import numpy as np
from jax import lax
from jax.experimental import pallas as pl
from jax.experimental.pallas import tpu as pltpu
from jax.experimental.pallas import tpu_sc as plsc

TOP_RATIO, MIN_PATCHES, RANDOM_RATIO = 0.7, 128, 0.1

_LANES = 16
_ROW_BLK = 1024
_GCHUNK = 64
_NWORKERS = 32


def _compute_sizes(n):
    k = min(max(int(n * TOP_RATIO), MIN_PATCHES), n)
    n_rand = max(int(k * RANDOM_RATIO), 1)
    return k, k - n_rand, n_rand


def _rand_tables(batch, n_nontop, n_rand, pad):
    rows = []
    for b in range(batch):
        perm = jax.random.permutation(
            jax.random.fold_in(jax.random.key(42), b), n_nontop)[:n_rand]
        rows.append(jnp.zeros((pad,), jnp.int32).at[perm].set(1))
    return jnp.stack(rows)


def _scores_body(x_ref, wv_ref, bv_ref, wu_ref, bu_ref, wa_ref, ba_ref, o_ref):
    x16 = x_ref[...].astype(jnp.bfloat16)
    a_v = jnp.tanh(jnp.dot(x16, wv_ref[...].astype(jnp.bfloat16),
                           preferred_element_type=jnp.float32) + bv_ref[...])
    a_u = jax.nn.sigmoid(jnp.dot(x16, wu_ref[...].astype(jnp.bfloat16),
                                 preferred_element_type=jnp.float32) + bu_ref[...])
    g16 = (a_v * a_u).astype(jnp.bfloat16)
    wa16 = wa_ref[...].astype(jnp.bfloat16)
    prod = g16.astype(jnp.float32) * wa16.astype(jnp.float32)
    o_ref[...] = jnp.sum(prod, axis=-1, keepdims=True) + ba_ref[...]


def _scores(features2d, Wv, bv, Wu, bu, w_att, b_att):
    bn, d = features2d.shape
    h = Wv.shape[1]
    grid = (bn // _ROW_BLK,)
    return pl.pallas_call(
        _scores_body,
        grid=grid,
        in_specs=[
            pl.BlockSpec((_ROW_BLK, d), lambda i: (i, 0)),
            pl.BlockSpec((d, h), lambda i: (0, 0)),
            pl.BlockSpec((1, h), lambda i: (0, 0)),
            pl.BlockSpec((d, h), lambda i: (0, 0)),
            pl.BlockSpec((1, h), lambda i: (0, 0)),
            pl.BlockSpec((1, h), lambda i: (0, 0)),
            pl.BlockSpec((1, 1), lambda i: (0, 0)),
        ],
        out_specs=pl.BlockSpec((_ROW_BLK, 1), lambda i: (i, 0)),
        out_shape=jax.ShapeDtypeStruct((bn, 1), jnp.float32),
    )(features2d, Wv, bv.reshape(1, h), Wu, bu.reshape(1, h),
      w_att.reshape(1, h), b_att.reshape(1, 1))


def _select_body(n_top, raw_ref, attn_ref, skey_ref, params_ref):
    raw = raw_ref[...]
    b = raw.shape[0]
    m = jnp.max(raw, axis=-1, keepdims=True)
    e = jnp.exp(raw - m)
    s = jnp.sum(e, axis=-1, keepdims=True)
    attn = e / s
    attn_ref[...] = attn

    u = lax.bitcast_convert_type(attn, jnp.uint32)
    sign = jnp.uint32(0x80000000)
    u = jnp.where(u >= sign, ~u, u | sign)

    def bit_step(t, p):
        sh = jnp.uint32(31) - t.astype(jnp.uint32)
        cand = p | (jnp.uint32(1) << sh)
        cnt = jnp.sum((u >= cand).astype(jnp.int32), axis=-1, keepdims=True)
        return jnp.where(cnt >= n_top, cand, p)

    p = lax.fori_loop(0, 32, bit_step, jnp.zeros((b, 1), jnp.uint32))
    n_greater = jnp.sum((u > p).astype(jnp.int32), axis=-1, keepdims=True)
    quota = jnp.full((b, 1), n_top, jnp.int32) - n_greater
    skey_ref[...] = lax.bitcast_convert_type(u ^ sign, jnp.int32)
    p_i = lax.bitcast_convert_type(p ^ sign, jnp.int32)
    params_ref[...] = jnp.concatenate(
        [p_i, quota, jnp.zeros((b, _LANES - 2), jnp.int32)], axis=1)


def _select(raw, n_top):
    b, n = raw.shape
    return pl.pallas_call(
        functools.partial(_select_body, n_top),
        out_shape=(
            jax.ShapeDtypeStruct((b, n), jnp.float32),
            jax.ShapeDtypeStruct((b, n), jnp.int32),
            jax.ShapeDtypeStruct((b, _LANES), jnp.int32),
        ),
    )(raw)


def _sc_compact(skey, params, rand_t, n, k, k_pad):
    b = skey.shape[0]
    rt_pad = rand_t.shape[1]
    mesh = plsc.VectorSubcoreMesh(core_axis_name="c", subcore_axis_name="s")

    @functools.partial(
        pl.kernel,
        mesh=mesh,
        out_type=(
            jax.ShapeDtypeStruct((b, k_pad), jnp.int32),
            jax.ShapeDtypeStruct((b * k_pad,), jnp.int32),
        ),
        compiler_params=pltpu.CompilerParams(needs_layout_passes=False),
        scratch_types=[
            pltpu.VMEM((n,), jnp.int32),
            pltpu.VMEM((rt_pad,), jnp.int32),
            pltpu.VMEM((_LANES,), jnp.int32),
            pltpu.VMEM((k_pad + _LANES,), jnp.int32),
            pltpu.VMEM((k_pad + _LANES,), jnp.int32),
        ],
    )
    def run(skey_hbm, params_hbm, rand_hbm, idx_hbm, gid_hbm,
            skey_v, rand_v, prm_v, sel_v, gid_v):
        wid = lax.axis_index("s") * 2 + lax.axis_index("c")

        @pl.when(wid < b)
        def _():
            pltpu.sync_copy(skey_hbm.at[wid], skey_v)
            pltpu.sync_copy(rand_hbm.at[wid], rand_v)
            pltpu.sync_copy(params_hbm.at[wid], prm_v)
            prm = prm_v[...]
            p_s = prm[0]
            quota = prm[1]
            b_n = wid * n
            for pad_base in range(k // _LANES * _LANES, k_pad, _LANES):
                sel_v[pl.ds(pad_base, _LANES)] = jnp.zeros((_LANES,), jnp.int32)
                gid_v[pl.ds(pad_base, _LANES)] = jnp.full((_LANES,), b_n, jnp.int32)

            def chunk(i, carry):
                off, rank_nt, tie_cnt = carry
                base = i * _LANES
                skv = skey_v[pl.ds(base, _LANES)]
                greater = skv > p_s
                tie = skv == p_s
                one, zero = jnp.int32(1), jnp.int32(0)
                ti = jnp.where(tie, one, zero)
                texcl = plsc.cumsum(ti) - ti
                tie_take = tie & ((texcl + tie_cnt) < quota)
                chosen = greater | tie_take
                nt = jnp.logical_not(chosen)
                nti = jnp.where(nt, one, zero)
                nexcl = plsc.cumsum(nti) - nti
                ridx = jnp.where(nt, rank_nt + nexcl, 0)
                rv = plsc.load_gather(rand_v, [ridx], mask=nt)
                sel = chosen | (nt & (rv > 0))
                ivals = base + lax.iota(jnp.int32, _LANES)
                plsc.store_compressed(sel_v.at[pl.ds(off, _LANES)], ivals, mask=sel)
                plsc.store_compressed(gid_v.at[pl.ds(off, _LANES)], ivals + b_n, mask=sel)
                n_sel = plsc.all_reduce_population_count(sel)[0]
                n_nt = plsc.all_reduce_population_count(nt)[0]
                n_tie = plsc.all_reduce_population_count(tie)[0]
                return off + n_sel, rank_nt + n_nt, tie_cnt + n_tie

            lax.fori_loop(0, n // _LANES, chunk,
                          (jnp.int32(0), jnp.int32(0), jnp.int32(0)))
            pltpu.sync_copy(sel_v.at[pl.ds(0, k_pad)], idx_hbm.at[wid])
            pltpu.sync_copy(gid_v.at[pl.ds(0, k_pad)],
                            gid_hbm.at[pl.ds(wid * k_pad, k_pad)])

    return run(skey, params, rand_t)


def _sc_gather(features2d, gid):
    m = gid.shape[0]
    d = features2d.shape[1]
    n_chunks = m // _GCHUNK
    per = -(-n_chunks // _NWORKERS)
    mesh = plsc.VectorSubcoreMesh(core_axis_name="c", subcore_axis_name="s")

    @functools.partial(
        pl.kernel,
        mesh=mesh,
        out_type=jax.ShapeDtypeStruct((m, d), jnp.float32),
        compiler_params=pltpu.CompilerParams(needs_layout_passes=False),
        scratch_types=[
            pltpu.VMEM((_GCHUNK,), jnp.int32),
            pltpu.VMEM((_GCHUNK, d), jnp.float32),
            pltpu.SemaphoreType.DMA,
        ],
    )
    def run(feat_hbm, gid_hbm, out_hbm, idx_v, rows_v, sem):
        wid = lax.axis_index("s") * 2 + lax.axis_index("c")
        lo = wid * per
        hi = jnp.minimum(lo + per, n_chunks)

        def body(ci, carry):
            base = ci * _GCHUNK
            pltpu.sync_copy(gid_hbm.at[pl.ds(base, _GCHUNK)], idx_v)
            pltpu.async_copy(feat_hbm.at[idx_v], rows_v, sem).wait()
            pltpu.sync_copy(rows_v, out_hbm.at[pl.ds(base, _GCHUNK)])
            return carry

        lax.fori_loop(lo, hi, body, jnp.int32(0))

    return run(features2d, gid)


def kernel(features, Wv, bv, Wu, bu, w_att, b_att):
    b, n, d = features.shape
    k, n_top, n_rand = _compute_sizes(n)
    k_pad = -(-k // 128) * 128
    n_nontop = n - n_top
    rt_pad = -(-n_nontop // _LANES) * _LANES

    feats2d = features.reshape(b * n, d)
    raw = _scores(feats2d, Wv, bv, Wu, bu, w_att, b_att).reshape(b, n)
    attn, skey, params = _select(raw, n_top)
    rand_t = _rand_tables(b, n_nontop, n_rand, rt_pad)
    idx_pad, gid = _sc_compact(skey, params, rand_t, n, k, k_pad)
    rows = _sc_gather(feats2d, gid)
    selected = rows.reshape(b, k_pad, d)[:, :k, :]
    return selected, attn, idx_pad[:, :k]
